# Initial kernel scaffold; baseline (speedup 1.0000x reference)
import jax
import jax.numpy as jnp
from jax.experimental import pallas as pl


def kernel(X, ppr_scores, W1, W2, W3, W4):
    raise NotImplementedError("write your pallas kernel here")


# speedup vs baseline: 3.4478x; 3.4478x over previous
import functools

import jax
import jax.numpy as jnp
from jax import lax
from jax.experimental import pallas as pl
from jax.experimental.pallas import tpu as pltpu

N = 4096
B = 1024
F = 128
H = 256
C = 64
K = 32

def _dot(a, b, trans_b=False):
    dn = (((1,), (1 if trans_b else 0,)), ((), ()))
    return lax.dot_general(a.astype(jnp.bfloat16), b.astype(jnp.bfloat16),
                           dn, preferred_element_type=jnp.float32)


def _emb_body(x_ref, w1_ref, w2_ref, emb_ref):
    h = jnp.maximum(_dot(x_ref[...], w1_ref[...]), 0.0)
    emb_ref[...] = _dot(h, w2_ref[...])


def _emb_call(X, W1, W2):
    blk = 512
    return pl.pallas_call(
        _emb_body,
        grid=(N // blk,),
        in_specs=[
            pl.BlockSpec((blk, F), lambda i: (i, 0)),
            pl.BlockSpec((F, H), lambda i: (0, 0)),
            pl.BlockSpec((H, H), lambda i: (0, 0)),
        ],
        out_specs=pl.BlockSpec((blk, H), lambda i: (i, 0)),
        out_shape=jax.ShapeDtypeStruct((N, H), jnp.float32),
    )(X, W1, W2)


_TI = 512
_TJ = 512


def _ad_body(ppr_ref, emb_ref, embi_ref, ad_ref):
    embI = embi_ref[...]
    sqI = jnp.sum(embI * embI, axis=1, keepdims=True)
    acc = jnp.zeros((B, _TI), jnp.float32)
    for jt in range(N // _TJ):
        embJ = emb_ref[jt * _TJ:(jt + 1) * _TJ, :]
        sqJ = jnp.sum(embJ * embJ, axis=1, keepdims=True)
        g = _dot(embJ, embI, trans_b=True)
        d2 = sqJ + sqI.T - 2.0 * g
        d2 = jnp.maximum(d2, 0.0)
        good = d2 > 1e-12
        dist = jnp.where(good, jnp.sqrt(jnp.where(good, d2, 1.0)), 0.0)
        a_j = ppr_ref[:, jt * _TJ:(jt + 1) * _TJ]
        acc = acc + _dot(a_j, dist)
    ad_ref[...] = acc


def _ad_call(ppr, emb):
    return pl.pallas_call(
        _ad_body,
        grid=(N // _TI,),
        in_specs=[
            pl.BlockSpec((B, N), lambda i: (0, 0)),
            pl.BlockSpec((N, H), lambda i: (0, 0)),
            pl.BlockSpec((_TI, H), lambda i: (i, 0)),
        ],
        out_specs=pl.BlockSpec((B, _TI), lambda i: (0, i)),
        out_shape=jax.ShapeDtypeStruct((B, N), jnp.float32),
    )(ppr, emb, emb)


_BR3 = 128


def _topk_body(ppr_ref, vals_ref, idx_ref, rs_ref):
    a = ppr_ref[...]
    rs_ref[...] = jnp.sum(a, axis=1, keepdims=True)
    iota = lax.broadcasted_iota(jnp.int32, (_BR3, N), 1)

    def step(c, work):
        m = jnp.max(work, axis=1, keepdims=True)
        cand = jnp.where(work == m, iota, N)
        j = jnp.min(cand, axis=1, keepdims=True)
        vals_ref[pl.ds(c, 1), :] = m.T
        idx_ref[pl.ds(c, 1), :] = j.T
        return jnp.where(iota == j, -1.0, work)

    lax.fori_loop(0, K, step, a)


def _topk_call(ppr):
    return pl.pallas_call(
        _topk_body,
        grid=(B // _BR3,),
        in_specs=[pl.BlockSpec((_BR3, N), lambda r: (r, 0))],
        out_specs=[
            pl.BlockSpec((K, _BR3), lambda r: (0, r)),
            pl.BlockSpec((K, _BR3), lambda r: (0, r)),
            pl.BlockSpec((_BR3, 1), lambda r: (r, 0)),
        ],
        out_shape=[
            jax.ShapeDtypeStruct((K, B), jnp.float32),
            jax.ShapeDtypeStruct((K, B), jnp.int32),
            jax.ShapeDtypeStruct((B, 1), jnp.float32),
        ],
    )(ppr)


_BR4 = 256


def _combine_body(ad_ref, vals_ref, idx_ref, rs_ref, emb_ref, w3_ref, w4_ref,
                  out_ref):
    ad = ad_ref[...]
    vals = vals_ref[...]
    idx = idx_ref[...]
    iota = lax.broadcasted_iota(jnp.int32, (_BR4, N), 1)

    rows = []
    for c in range(K):
        mask = iota == idx[c, :][:, None]
        rows.append(jnp.sum(jnp.where(mask, ad, 0.0), axis=1, keepdims=True).T)
    ds = jnp.concatenate(rows, axis=0)

    ds = jnp.where(vals == 0.0, 1e30, ds)
    neg = -ds
    m = jnp.max(neg, axis=0, keepdims=True)
    e = jnp.exp(neg - m)
    sm = e / jnp.sum(e, axis=0, keepdims=True)
    sm = sm * vals
    sm = sm / jnp.sum(sm, axis=0, keepdims=True)

    w = jnp.zeros((_BR4, N), jnp.float32)
    for c in range(K):
        mask = iota == idx[c, :][:, None]
        w = w + jnp.where(mask, sm[c, :][:, None], 0.0)

    agg = rs_ref[...] * _dot(w, emb_ref[...])
    h = jnp.maximum(_dot(agg, w3_ref[...]), 0.0)
    out_ref[...] = _dot(h, w4_ref[...])


def _combine_call(ad, vals, idx, rs, emb, W3, W4):
    return pl.pallas_call(
        _combine_body,
        grid=(B // _BR4,),
        in_specs=[
            pl.BlockSpec((_BR4, N), lambda r: (r, 0)),
            pl.BlockSpec((K, _BR4), lambda r: (0, r)),
            pl.BlockSpec((K, _BR4), lambda r: (0, r)),
            pl.BlockSpec((_BR4, 1), lambda r: (r, 0)),
            pl.BlockSpec((N, H), lambda r: (0, 0)),
            pl.BlockSpec((H, H), lambda r: (0, 0)),
            pl.BlockSpec((H, C), lambda r: (0, 0)),
        ],
        out_specs=pl.BlockSpec((_BR4, C), lambda r: (r, 0)),
        out_shape=jax.ShapeDtypeStruct((B, C), jnp.float32),
    )(ad, vals, idx, rs, emb, W3, W4)


def kernel(X, ppr_scores, W1, W2, W3, W4):
    emb = _emb_call(X, W1, W2)
    ad = _ad_call(ppr_scores, emb)
    vals, idx, rs = _topk_call(ppr_scores)
    return _combine_call(ad, vals, idx, rs, emb, W3, W4)


# speedup vs baseline: 8.8618x; 2.5703x over previous
import functools

import jax
import jax.numpy as jnp
from jax import lax
from jax.experimental import pallas as pl
from jax.experimental.pallas import tpu as pltpu

N = 4096
B = 1024
F = 128
H = 256
C = 64
K = 32

def _dot(a, b, trans_b=False):
    dn = (((1,), (1 if trans_b else 0,)), ((), ()))
    return lax.dot_general(a.astype(jnp.bfloat16), b.astype(jnp.bfloat16),
                           dn, preferred_element_type=jnp.float32)


def _emb_body(x_ref, w1_ref, w2_ref, emb_ref):
    h = jnp.maximum(_dot(x_ref[...], w1_ref[...]), 0.0)
    emb_ref[...] = _dot(h, w2_ref[...])


def _emb_call(X, W1, W2):
    blk = 512
    return pl.pallas_call(
        _emb_body,
        grid=(N // blk,),
        in_specs=[
            pl.BlockSpec((blk, F), lambda i: (i, 0)),
            pl.BlockSpec((F, H), lambda i: (0, 0)),
            pl.BlockSpec((H, H), lambda i: (0, 0)),
        ],
        out_specs=pl.BlockSpec((blk, H), lambda i: (i, 0)),
        out_shape=jax.ShapeDtypeStruct((N, H), jnp.float32),
    )(X, W1, W2)


_TI = 512
_TJ = 512


def _ad_body(ppr_ref, emb_ref, embi_ref, ad_ref):
    embI = embi_ref[...]
    sqI = jnp.sum(embI * embI, axis=1, keepdims=True)
    acc = jnp.zeros((B, _TI), jnp.float32)
    for jt in range(N // _TJ):
        embJ = emb_ref[jt * _TJ:(jt + 1) * _TJ, :]
        sqJ = jnp.sum(embJ * embJ, axis=1, keepdims=True)
        g = _dot(embJ, embI, trans_b=True)
        d2 = sqJ + sqI.T - 2.0 * g
        d2 = jnp.maximum(d2, 0.0)
        good = d2 > 1e-12
        dist = jnp.where(good, jnp.sqrt(jnp.where(good, d2, 1.0)), 0.0)
        a_j = ppr_ref[:, jt * _TJ:(jt + 1) * _TJ]
        acc = acc + _dot(a_j, dist)
    ad_ref[...] = acc


def _ad_call(ppr, emb):
    return pl.pallas_call(
        _ad_body,
        grid=(N // _TI,),
        in_specs=[
            pl.BlockSpec((B, N), lambda i: (0, 0)),
            pl.BlockSpec((N, H), lambda i: (0, 0)),
            pl.BlockSpec((_TI, H), lambda i: (i, 0)),
        ],
        out_specs=pl.BlockSpec((B, _TI), lambda i: (0, i)),
        out_shape=jax.ShapeDtypeStruct((B, N), jnp.float32),
    )(ppr, emb, emb)


_BR3 = 256


def _sel_body(ppr_ref, sel_ref, rs_ref):
    a = ppr_ref[...]
    rs_ref[...] = jnp.sum(a, axis=1, keepdims=True)
    bits = pltpu.bitcast(a, jnp.int32)

    def bstep(it, t):
        cand = t | lax.shift_left(1, 29 - it)
        ge = bits >= cand
        cnt = jnp.sum(jnp.where(ge, 1.0, 0.0), axis=1, keepdims=True)
        return jnp.where(cnt >= float(K), cand, t)

    t0 = jnp.zeros((_BR3, 1), jnp.int32)
    t = lax.fori_loop(0, 30, bstep, t0)

    gt = bits > t
    eq = bits == t
    c_gt = jnp.sum(jnp.where(gt, 1.0, 0.0), axis=1, keepdims=True)
    c_eq = jnp.sum(jnp.where(eq, 1.0, 0.0), axis=1, keepdims=True)
    need = float(K) - c_gt
    iota = lax.broadcasted_iota(jnp.int32, (_BR3, N), 1)

    take_all = c_eq <= need
    sel_eq = jnp.where(take_all & eq, 1.0, 0.0)
    rem = jnp.where(take_all, 0.0, need)

    def cond(carry):
        sel_eq, rem = carry
        return jnp.max(rem) > 0.0

    def pick(carry):
        sel_eq, rem = carry
        open_row = rem > 0.0
        pickable = eq & (sel_eq == 0.0) & open_row
        j = jnp.min(jnp.where(pickable, iota, N), axis=1, keepdims=True)
        sel_eq = jnp.where((iota == j) & open_row, 1.0, sel_eq)
        return sel_eq, jnp.maximum(rem - 1.0, 0.0)

    sel_eq, _ = lax.while_loop(cond, pick, (sel_eq, rem))
    sel_ref[...] = jnp.where(gt, 1.0, sel_eq)


def _sel_call(ppr):
    return pl.pallas_call(
        _sel_body,
        grid=(B // _BR3,),
        in_specs=[pl.BlockSpec((_BR3, N), lambda r: (r, 0))],
        out_specs=[
            pl.BlockSpec((_BR3, N), lambda r: (r, 0)),
            pl.BlockSpec((_BR3, 1), lambda r: (r, 0)),
        ],
        out_shape=[
            jax.ShapeDtypeStruct((B, N), jnp.float32),
            jax.ShapeDtypeStruct((B, 1), jnp.float32),
        ],
    )(ppr)


_BR4 = 256


def _combine_body(ad_ref, sel_ref, ppr_ref, rs_ref, emb_ref, w3_ref, w4_ref,
                  out_ref):
    ad = ad_ref[...]
    sel = sel_ref[...] > 0.0
    a = ppr_ref[...]

    z = jnp.where(sel, jnp.where(a > 0.0, -ad, -1e30), -jnp.inf)
    m = jnp.max(z, axis=1, keepdims=True)
    e = jnp.exp(z - m)
    sm = e / jnp.sum(e, axis=1, keepdims=True)
    sm = sm * a
    sm = sm / jnp.sum(sm, axis=1, keepdims=True)

    agg = rs_ref[...] * _dot(sm, emb_ref[...])
    h = jnp.maximum(_dot(agg, w3_ref[...]), 0.0)
    out_ref[...] = _dot(h, w4_ref[...])


def _combine_call(ad, sel, ppr, rs, emb, W3, W4):
    return pl.pallas_call(
        _combine_body,
        grid=(B // _BR4,),
        in_specs=[
            pl.BlockSpec((_BR4, N), lambda r: (r, 0)),
            pl.BlockSpec((_BR4, N), lambda r: (r, 0)),
            pl.BlockSpec((_BR4, N), lambda r: (r, 0)),
            pl.BlockSpec((_BR4, 1), lambda r: (r, 0)),
            pl.BlockSpec((N, H), lambda r: (0, 0)),
            pl.BlockSpec((H, H), lambda r: (0, 0)),
            pl.BlockSpec((H, C), lambda r: (0, 0)),
        ],
        out_specs=pl.BlockSpec((_BR4, C), lambda r: (r, 0)),
        out_shape=jax.ShapeDtypeStruct((B, C), jnp.float32),
    )(ad, sel, ppr, rs, emb, W3, W4)


def kernel(X, ppr_scores, W1, W2, W3, W4):
    emb = _emb_call(X, W1, W2)
    ad = _ad_call(ppr_scores, emb)
    sel, rs = _sel_call(ppr_scores)
    return _combine_call(ad, sel, ppr_scores, rs, emb, W3, W4)
